# baseline (speedup 1.0000x reference)
import functools

import jax
import jax.numpy as jnp
from jax import lax
from jax.experimental import pallas as pl
from jax.experimental.pallas import tpu as pltpu
from jax.experimental.pallas import tpu_sc as plsc

B = 4096
L = 200
D = 32
NC = 2
NS = 16
NW = NC * NS
SEQ_PER_W = B // NW
SPLIT = 104
SPLIT2 = L - SPLIT
NBUF = 4
GROUPS = SEQ_PER_W // NBUF

LBLK = 8
BBLK = 128


def _sc_gather(seq_hbm, tok_hbm, out_hbm, idx_v, *bufs):
    rows = list(bufs[0:NBUF])
    gsem = list(bufs[NBUF:2 * NBUF])
    osem = list(bufs[2 * NBUF:3 * NBUF])
    wid = lax.axis_index("s") * NC + lax.axis_index("c")
    base = wid * SEQ_PER_W

    pltpu.sync_copy(seq_hbm.at[pl.ds(base, SEQ_PER_W)], idx_v)

    def gfire(s, b):
        pltpu.async_copy(tok_hbm.at[idx_v.at[s, pl.ds(0, SPLIT)]],
                         rows[b].at[pl.ds(0, SPLIT)], gsem[b])
        pltpu.async_copy(tok_hbm.at[idx_v.at[s, pl.ds(SPLIT, SPLIT2)]],
                         rows[b].at[pl.ds(SPLIT, SPLIT2)], gsem[b])

    def gwait(b):
        pltpu.make_async_copy(tok_hbm.at[idx_v.at[0, pl.ds(0, SPLIT)]],
                              rows[b].at[pl.ds(0, SPLIT)], gsem[b]).wait()
        pltpu.make_async_copy(tok_hbm.at[idx_v.at[0, pl.ds(SPLIT, SPLIT2)]],
                              rows[b].at[pl.ds(SPLIT, SPLIT2)], gsem[b]).wait()

    for b in range(NBUF):
        gfire(b, b)

    def group(g, carry):
        for b in range(NBUF):
            gwait(b)
            pltpu.async_copy(rows[b], out_hbm.at[base + g * NBUF + b], osem[b])
        for b in range(NBUF):
            pltpu.make_async_copy(rows[b], out_hbm.at[base], osem[b]).wait()

            @pl.when(g < GROUPS - 1)
            def _fire_next():
                gfire((g + 1) * NBUF + b, b)

        return carry

    lax.fori_loop(0, GROUPS, group, 0)


def _tc_body(g_ref, pos_ref, out_ref):
    for i in range(LBLK):
        blk = g_ref[:, i, :]
        out_ref[i] = blk.T + pos_ref[i][:, None]


def kernel(seq, token_table, pos_table):
    mesh = plsc.VectorSubcoreMesh(
        core_axis_name="c", subcore_axis_name="s",
        num_cores=NC, num_subcores=NS)
    scratch = [pltpu.VMEM((SEQ_PER_W, L), jnp.int32)]
    scratch += [pltpu.VMEM((L, D), jnp.float32) for _ in range(NBUF)]
    scratch += [pltpu.SemaphoreType.DMA for _ in range(2 * NBUF)]
    sc = functools.partial(
        pl.kernel,
        out_type=jax.ShapeDtypeStruct((B, L, D), jnp.float32),
        mesh=mesh,
        scratch_types=scratch,
        compiler_params=pltpu.CompilerParams(use_tc_tiling_on_sc=False),
    )(_sc_gather)
    g3 = sc(seq, token_table)

    tc = pl.pallas_call(
        _tc_body,
        grid=(L // LBLK, B // BBLK),
        in_specs=[
            pl.BlockSpec((BBLK, LBLK, D), lambda l, b: (b, l, 0)),
            pl.BlockSpec((LBLK, D), lambda l, b: (l, 0)),
        ],
        out_specs=pl.BlockSpec((LBLK, D, BBLK), lambda l, b: (l, 0, b)),
        out_shape=jax.ShapeDtypeStruct((L, D, B), jnp.float32),
    )
    out_p = tc(g3, pos_table)
    return out_p.transpose(2, 0, 1)


# speedup vs baseline: 1.0172x; 1.0172x over previous
import functools

import jax
import jax.numpy as jnp
from jax import lax
from jax.experimental import pallas as pl
from jax.experimental.pallas import tpu as pltpu
from jax.experimental.pallas import tpu_sc as plsc

B = 4096
L = 200
D = 32
NC = 2
NS = 16
NW = NC * NS
BW = B // NW
NBUF = 4
GROUPS = L // NBUF

LBLK = 8


def _sc_gather(seq_hbm, tok_hbm, out_hbm, seq_v, *bufs):
    rows = list(bufs[0:NBUF])
    gsem = list(bufs[NBUF:2 * NBUF])
    osem = list(bufs[2 * NBUF:3 * NBUF])
    wid = lax.axis_index("s") * NC + lax.axis_index("c")

    pltpu.sync_copy(seq_hbm.at[:, wid], seq_v)

    def gfire(l, b):
        pltpu.async_copy(tok_hbm.at[seq_v.at[l // 8, l % 8]], rows[b], gsem[b])

    def gwait(b):
        pltpu.make_async_copy(tok_hbm.at[seq_v.at[0, 0]], rows[b],
                              gsem[b]).wait()

    for b in range(NBUF):
        gfire(b, b)

    def group(g, carry):
        for b in range(NBUF):
            l = g * NBUF + b
            gwait(b)
            pltpu.async_copy(rows[b], out_hbm.at[l, wid], osem[b])
        for b in range(NBUF):
            pltpu.make_async_copy(rows[b], out_hbm.at[0, wid], osem[b]).wait()

            @pl.when(g < GROUPS - 1)
            def _fire_next():
                gfire((g + 1) * NBUF + b, b)

        return carry

    lax.fori_loop(0, GROUPS, group, 0)


def _tc_body(g_ref, pos_ref, out_ref):
    for i in range(LBLK):
        blk = g_ref[i, 0]
        out_ref[i] = blk.T + pos_ref[i][:, None]


def kernel(seq, token_table, pos_table):
    mesh = plsc.VectorSubcoreMesh(
        core_axis_name="c", subcore_axis_name="s",
        num_cores=NC, num_subcores=NS)
    scratch = [pltpu.VMEM((L // 8, 8, BW), jnp.int32)]
    scratch += [pltpu.VMEM((BW, D), jnp.float32) for _ in range(NBUF)]
    scratch += [pltpu.SemaphoreType.DMA for _ in range(2 * NBUF)]
    sc = functools.partial(
        pl.kernel,
        out_type=jax.ShapeDtypeStruct((L, NW, BW, D), jnp.float32),
        mesh=mesh,
        scratch_types=scratch,
        compiler_params=pltpu.CompilerParams(use_tc_tiling_on_sc=False),
    )(_sc_gather)

    seq4 = seq.T.reshape(L // 8, 8, NW, BW).transpose(0, 2, 1, 3)
    g4 = sc(seq4, token_table)

    tc = pl.pallas_call(
        _tc_body,
        grid=(L // LBLK, NW),
        in_specs=[
            pl.BlockSpec((LBLK, 1, BW, D), lambda l, c: (l, c, 0, 0)),
            pl.BlockSpec((LBLK, D), lambda l, c: (l, 0)),
        ],
        out_specs=pl.BlockSpec((LBLK, D, BW), lambda l, c: (l, 0, c)),
        out_shape=jax.ShapeDtypeStruct((L, D, B), jnp.float32),
    )
    out_p = tc(g4, pos_table)
    return out_p.transpose(2, 0, 1)


# speedup vs baseline: 1.8388x; 1.8077x over previous
import functools

import jax
import jax.numpy as jnp
from jax import lax
from jax.experimental import pallas as pl
from jax.experimental.pallas import tpu as pltpu
from jax.experimental.pallas import tpu_sc as plsc

B = 4096
L = 200
D = 32
NC = 2
NS = 16
NW = NC * NS
BW = B // NW
LANES = 16
NBUF = 4
GROUPS = L // NBUF


def _body(seq_hbm, tok_hbm, pos_hbm, out_hbm, seq_v, pos_v, *bufs):
    rows = list(bufs[0:NBUF])
    trans = list(bufs[NBUF:2 * NBUF])
    gsem = list(bufs[2 * NBUF:3 * NBUF])
    osem = list(bufs[3 * NBUF:4 * NBUF])
    wid = lax.axis_index("s") * NC + lax.axis_index("c")

    pltpu.sync_copy(seq_hbm.at[:, wid], seq_v)
    pltpu.sync_copy(pos_hbm, pos_v)

    def gfire(l, b):
        pltpu.async_copy(tok_hbm.at[seq_v.at[l // 8, l % 8]], rows[b], gsem[b])

    def gwait(b):
        pltpu.make_async_copy(tok_hbm.at[seq_v.at[0, 0]], rows[b],
                              gsem[b]).wait()

    iota = lax.iota(jnp.int32, LANES)

    def transpose_add(l, b):
        rb = rows[b]
        tb = trans[b]

        @plsc.parallel_loop(0, D, 1, unroll=4)
        def _(f):
            fsplat = iota * 0 + f
            psplat = plsc.load_gather(pos_v, [iota * 0 + l, fsplat])
            for g in range(BW // LANES):
                bvec = iota + (g * LANES)
                v = plsc.load_gather(rb, [bvec, fsplat]) + psplat
                tb[f // 8, f % 8, pl.ds(g * LANES, LANES)] = v

    for b in range(NBUF):
        gfire(b, b)

    def group(g, carry):
        for b in range(NBUF):
            l = g * NBUF + b
            gwait(b)
            transpose_add(l, b)
            pltpu.async_copy(trans[b], out_hbm.at[l, :, wid], osem[b])
        for b in range(NBUF):
            pltpu.make_async_copy(
                trans[b], out_hbm.at[0, :, wid], osem[b]).wait()

            @pl.when(g < GROUPS - 1)
            def _fire_next():
                gfire((g + 1) * NBUF + b, b)

        return carry

    lax.fori_loop(0, GROUPS, group, 0)


TCOLS = 4096


def _tc_table(tt_ref, out_ref):
    x = tt_ref[...]
    xt = x.T.reshape(TCOLS // 4, 4, D)
    for q in range(4):
        out_ref[:, q * D:(q + 1) * D] = xt[:, q, :]


def kernel(seq, token_table, pos_table):
    mesh = plsc.VectorSubcoreMesh(
        core_axis_name="c", subcore_axis_name="s",
        num_cores=NC, num_subcores=NS)
    scratch = [
        pltpu.VMEM((L // 8, 8, BW), jnp.int32),
        pltpu.VMEM((L, D), jnp.float32),
    ]
    scratch += [pltpu.VMEM((BW, D), jnp.float32) for _ in range(NBUF)]
    scratch += [pltpu.VMEM((D // 8, 8, BW), jnp.float32) for _ in range(NBUF)]
    scratch += [pltpu.SemaphoreType.DMA for _ in range(2 * NBUF)]
    k = functools.partial(
        pl.kernel,
        out_type=jax.ShapeDtypeStruct((L, D // 8, NW, 8, BW), jnp.float32),
        mesh=mesh,
        scratch_types=scratch,
        compiler_params=pltpu.CompilerParams(
            use_tc_tiling_on_sc=False, needs_layout_passes=False),
    )(_body)
    V = token_table.shape[0]
    tc_table = pl.pallas_call(
        _tc_table,
        grid=((V + TCOLS - 1) // TCOLS,),
        in_specs=[pl.BlockSpec((D, TCOLS), lambda i: (0, i))],
        out_specs=pl.BlockSpec((TCOLS // 4, 128), lambda i: (i, 0)),
        out_shape=jax.ShapeDtypeStruct((V // 4, 128), jnp.float32),
    )
    tok_lin = tc_table(token_table.T).reshape(V, D)

    seq4 = seq.T.reshape(L // 8, 8, NW, BW).transpose(0, 2, 1, 3)
    out6 = k(seq4, tok_lin, pos_table)
    return out6.transpose(2, 4, 0, 1, 3).reshape(B, L, D)
